# baseline (speedup 1.0000x reference)
import functools
import math

import jax
import jax.numpy as jnp
from jax import lax
from jax.experimental import pallas as pl
from jax.experimental.pallas import tpu as pltpu
from jax.experimental.pallas import tpu_sc as plsc

_BATCH = 4096
_FEATURES = 26
_MODELS = 128
_EPC = 1
_ROWS = _EPC * _FEATURES
_LANES = 16
_NLC = _MODELS // _LANES
_NBUF = 2


def _make_kernel(num_cores, num_subcores):
    nw = num_cores * num_subcores
    bpw = _BATCH // nw
    nchunk = bpw // _EPC
    mesh = plsc.VectorSubcoreMesh(core_axis_name="c", subcore_axis_name="s")

    @functools.partial(
        pl.kernel,
        out_type=jax.ShapeDtypeStruct((_BATCH, 2, _MODELS), jnp.float32),
        mesh=mesh,
        scratch_types=[
            pltpu.VMEM((nchunk, _ROWS), jnp.int32),
            pltpu.VMEM((2, _FEATURES, _LANES), jnp.float32),
            pltpu.VMEM((2, _LANES), jnp.float32),
            [pltpu.VMEM((_ROWS, _MODELS), jnp.float32)] * _NBUF,
            [pltpu.VMEM((_EPC, 2, _MODELS), jnp.float32)] * _NBUF,
            [pltpu.SemaphoreType.DMA] * _NBUF,
            [pltpu.SemaphoreType.DMA] * _NBUF,
        ],
    )
    def k(table, idx, wv, bv, out, idx_v, w_v, b_v, rows, obufs, gsems,
          osems):
        wid = lax.axis_index("s") * num_cores + lax.axis_index("c")

        pltpu.sync_copy(idx.at[wid], idx_v)
        pltpu.sync_copy(wv, w_v)
        pltpu.sync_copy(bv, b_v)

        for s in range(_NBUF):
            pltpu.async_copy(table.at[idx_v.at[s]], rows[s], gsems[s])

        out_base = wid * bpw

        @pl.loop(0, nchunk, step=_NBUF)
        def _(j):
            for s in range(_NBUF):
                jj = j + s
                rbuf = rows[s]
                obuf = obufs[s]
                pltpu.make_async_copy(table.at[idx_v.at[jj]], rbuf,
                                      gsems[s]).wait()
                @pl.when(j > 0)
                def _():
                    pltpu.make_async_copy(
                        obuf, out.at[pl.ds(0, _EPC)], osems[s]).wait()

                bvec0 = b_v[0, :]
                bvec1 = b_v[1, :]

                ebs = min(2, _EPC)
                for eb in range(_EPC // ebs):
                    acc = [[[None] * _NLC for _ in range(ebs)]
                           for _ in range(2)]
                    for l in range(_FEATURES):
                        w0l = w_v[0, l, :]
                        w1l = w_v[1, l, :]
                        for e2 in range(ebs):
                            r = (eb * ebs + e2) * _FEATURES + l
                            for c in range(_NLC):
                                sl = pl.ds(c * _LANES, _LANES)
                                v = rbuf[r, sl]
                                if l == 0:
                                    acc[0][e2][c] = bvec0 + w0l * v
                                    acc[1][e2][c] = bvec1 + w1l * v
                                else:
                                    acc[0][e2][c] = acc[0][e2][c] + w0l * v
                                    acc[1][e2][c] = acc[1][e2][c] + w1l * v
                    for e2 in range(ebs):
                        for o in range(2):
                            for c in range(_NLC):
                                sl = pl.ds(c * _LANES, _LANES)
                                obuf[eb * ebs + e2, o, sl] = acc[o][e2][c]

                @pl.when(jj + _NBUF < nchunk)
                def _():
                    pltpu.async_copy(table.at[idx_v.at[jj + _NBUF]], rbuf,
                                     gsems[s])

                pltpu.async_copy(
                    obuf, out.at[pl.ds(out_base + jj * _EPC, _EPC)],
                    osems[s])

        for s in range(_NBUF):
            pltpu.make_async_copy(
                obufs[s], out.at[pl.ds(0, _EPC)], osems[s]).wait()

    return k


@jax.jit
def kernel(input, table, W, b):
    info = plsc.get_sparse_core_info()
    nw = info.num_cores * info.num_subcores
    idx = input.astype(jnp.int32).reshape(nw, (_BATCH // nw) // _EPC, _ROWS)
    scale = math.sqrt(float(_MODELS))
    wv = jnp.broadcast_to(
        (W.astype(jnp.float32).T * scale)[:, :, None],
        (2, _FEATURES, _LANES))
    bv = jnp.broadcast_to(b.astype(jnp.float32)[:, None], (2, _LANES))
    k = _make_kernel(info.num_cores, info.num_subcores)
    return k(table, idx, wv, bv)


# speedup vs baseline: 1.2999x; 1.2999x over previous
import functools
import math

import jax
import jax.numpy as jnp
from jax import lax
from jax.experimental import pallas as pl
from jax.experimental.pallas import tpu as pltpu
from jax.experimental.pallas import tpu_sc as plsc

_BATCH = 4096
_FEATURES = 26
_MODELS = 128
_EPC = 2
_ROWS = _EPC * _FEATURES
_LANES = 16
_NLC = _MODELS // _LANES
_NBUF = 2


def _make_kernel(num_cores, num_subcores):
    nw = num_cores * num_subcores
    bpw = _BATCH // nw
    nchunk = bpw // _EPC
    mesh = plsc.VectorSubcoreMesh(core_axis_name="c", subcore_axis_name="s")

    @functools.partial(
        pl.kernel,
        out_type=jax.ShapeDtypeStruct((_BATCH, 2, _MODELS), jnp.float32),
        mesh=mesh,
        scratch_types=[
            pltpu.VMEM((nchunk, _ROWS), jnp.int32),
            pltpu.VMEM((2, _FEATURES, _LANES), jnp.float32),
            pltpu.VMEM((2, _LANES), jnp.float32),
            [pltpu.VMEM((_ROWS, _MODELS), jnp.float32)] * _NBUF,
            [pltpu.VMEM((_EPC, 2, _MODELS), jnp.float32)] * _NBUF,
            [pltpu.SemaphoreType.DMA] * _NBUF,
            [pltpu.SemaphoreType.DMA] * _NBUF,
        ],
    )
    def k(table, idx, wv, bv, out, idx_v, w_v, b_v, rows, obufs, gsems,
          osems):
        wid = lax.axis_index("s") * num_cores + lax.axis_index("c")

        pltpu.async_copy(idx.at[wid], idx_v, osems[0])
        pltpu.async_copy(wv, w_v, osems[1])
        pltpu.async_copy(bv, b_v, osems[1])
        pltpu.make_async_copy(idx.at[wid], idx_v, osems[0]).wait()

        for s in range(_NBUF):
            pltpu.async_copy(table.at[idx_v.at[s]], rows[s], gsems[s])

        pltpu.make_async_copy(wv, w_v, osems[1]).wait()
        pltpu.make_async_copy(bv, b_v, osems[1]).wait()

        out_base = wid * bpw

        @pl.loop(0, nchunk, step=_NBUF)
        def _(j):
            for s in range(_NBUF):
                jj = j + s
                rbuf = rows[s]
                obuf = obufs[s]
                pltpu.make_async_copy(table.at[idx_v.at[jj]], rbuf,
                                      gsems[s]).wait()
                @pl.when(j > 0)
                def _():
                    pltpu.make_async_copy(
                        obuf, out.at[pl.ds(0, _EPC)], osems[s]).wait()

                bvec0 = b_v[0, :]
                bvec1 = b_v[1, :]

                ebs = min(2, _EPC)
                for eb in range(_EPC // ebs):
                    acc = [[[None] * _NLC for _ in range(ebs)]
                           for _ in range(2)]
                    for l in range(_FEATURES):
                        w0l = w_v[0, l, :]
                        w1l = w_v[1, l, :]
                        for e2 in range(ebs):
                            r = (eb * ebs + e2) * _FEATURES + l
                            for c in range(_NLC):
                                sl = pl.ds(c * _LANES, _LANES)
                                v = rbuf[r, sl]
                                if l == 0:
                                    acc[0][e2][c] = bvec0 + w0l * v
                                    acc[1][e2][c] = bvec1 + w1l * v
                                else:
                                    acc[0][e2][c] = acc[0][e2][c] + w0l * v
                                    acc[1][e2][c] = acc[1][e2][c] + w1l * v
                    for e2 in range(ebs):
                        for o in range(2):
                            for c in range(_NLC):
                                sl = pl.ds(c * _LANES, _LANES)
                                obuf[eb * ebs + e2, o, sl] = acc[o][e2][c]

                @pl.when(jj + _NBUF < nchunk)
                def _():
                    pltpu.async_copy(table.at[idx_v.at[jj + _NBUF]], rbuf,
                                     gsems[s])

                pltpu.async_copy(
                    obuf, out.at[pl.ds(out_base + jj * _EPC, _EPC)],
                    osems[s])

        for s in range(_NBUF):
            pltpu.make_async_copy(
                obufs[s], out.at[pl.ds(0, _EPC)], osems[s]).wait()

    return k


@jax.jit
def kernel(input, table, W, b):
    info = plsc.get_sparse_core_info()
    nw = info.num_cores * info.num_subcores
    idx = input.astype(jnp.int32).reshape(nw, (_BATCH // nw) // _EPC, _ROWS)
    scale = math.sqrt(float(_MODELS))
    wv = jnp.broadcast_to(
        (W.astype(jnp.float32).T * scale)[:, :, None],
        (2, _FEATURES, _LANES))
    bv = jnp.broadcast_to(b.astype(jnp.float32)[:, None], (2, _LANES))
    k = _make_kernel(info.num_cores, info.num_subcores)
    return k(table, idx, wv, bv)


# speedup vs baseline: 1.3083x; 1.0064x over previous
import functools
import math

import jax
import jax.numpy as jnp
from jax import lax
from jax.experimental import pallas as pl
from jax.experimental.pallas import tpu as pltpu
from jax.experimental.pallas import tpu_sc as plsc

_BATCH = 4096
_FEATURES = 26
_MODELS = 128
_EPC = 2
_ROWS = _EPC * _FEATURES
_LANES = 16
_NLC = _MODELS // _LANES
_NBUF = 2


def _make_kernel(num_cores, num_subcores):
    nw = num_cores * num_subcores
    bpw = _BATCH // nw
    nchunk = bpw // _EPC
    mesh = plsc.VectorSubcoreMesh(core_axis_name="c", subcore_axis_name="s")

    @functools.partial(
        pl.kernel,
        out_type=jax.ShapeDtypeStruct((_BATCH, 2, _MODELS), jnp.float32),
        mesh=mesh,
        scratch_types=[
            pltpu.VMEM((nchunk, _ROWS), jnp.int32),
            pltpu.VMEM((2, _FEATURES + 1, _LANES), jnp.float32),
            [pltpu.VMEM((_ROWS, _MODELS), jnp.float32)] * _NBUF,
            [pltpu.VMEM((_EPC, 2, _MODELS), jnp.float32)] * _NBUF,
            [pltpu.SemaphoreType.DMA] * _NBUF,
            [pltpu.SemaphoreType.DMA] * _NBUF,
        ],
    )
    def k(table, idx, wb, out, idx_v, w_v, rows, obufs, gsems,
          osems):
        wid = lax.axis_index("s") * num_cores + lax.axis_index("c")

        pltpu.async_copy(idx.at[wid], idx_v, osems[0])
        pltpu.async_copy(wb, w_v, osems[1])
        pltpu.make_async_copy(idx.at[wid], idx_v, osems[0]).wait()

        for s in range(_NBUF):
            pltpu.async_copy(table.at[idx_v.at[s]], rows[s], gsems[s])

        pltpu.make_async_copy(wb, w_v, osems[1]).wait()

        out_base = wid * bpw

        @pl.loop(0, nchunk, step=_NBUF)
        def _(j):
            for s in range(_NBUF):
                jj = j + s
                rbuf = rows[s]
                obuf = obufs[s]
                pltpu.make_async_copy(table.at[idx_v.at[jj]], rbuf,
                                      gsems[s]).wait()
                @pl.when(j > 0)
                def _():
                    pltpu.make_async_copy(
                        obuf, out.at[pl.ds(0, _EPC)], osems[s]).wait()

                bvec0 = w_v[0, _FEATURES, :]
                bvec1 = w_v[1, _FEATURES, :]

                ebs = min(2, _EPC)
                for eb in range(_EPC // ebs):
                    acc = [[[None] * _NLC for _ in range(ebs)]
                           for _ in range(2)]
                    for l in range(_FEATURES):
                        w0l = w_v[0, l, :]
                        w1l = w_v[1, l, :]
                        for e2 in range(ebs):
                            r = (eb * ebs + e2) * _FEATURES + l
                            for c in range(_NLC):
                                sl = pl.ds(c * _LANES, _LANES)
                                v = rbuf[r, sl]
                                if l == 0:
                                    acc[0][e2][c] = bvec0 + w0l * v
                                    acc[1][e2][c] = bvec1 + w1l * v
                                else:
                                    acc[0][e2][c] = acc[0][e2][c] + w0l * v
                                    acc[1][e2][c] = acc[1][e2][c] + w1l * v
                    for e2 in range(ebs):
                        for o in range(2):
                            for c in range(_NLC):
                                sl = pl.ds(c * _LANES, _LANES)
                                obuf[eb * ebs + e2, o, sl] = acc[o][e2][c]

                @pl.when(jj + _NBUF < nchunk)
                def _():
                    pltpu.async_copy(table.at[idx_v.at[jj + _NBUF]], rbuf,
                                     gsems[s])

                pltpu.async_copy(
                    obuf, out.at[pl.ds(out_base + jj * _EPC, _EPC)],
                    osems[s])

        for s in range(_NBUF):
            pltpu.make_async_copy(
                obufs[s], out.at[pl.ds(0, _EPC)], osems[s]).wait()

    return k


@jax.jit
def kernel(input, table, W, b):
    info = plsc.get_sparse_core_info()
    nw = info.num_cores * info.num_subcores
    idx = input.astype(jnp.int32).reshape(nw, (_BATCH // nw) // _EPC, _ROWS)
    scale = math.sqrt(float(_MODELS))
    wb = jnp.broadcast_to(
        jnp.concatenate(
            [W.astype(jnp.float32).T * scale,
             b.astype(jnp.float32)[:, None]], axis=1)[:, :, None],
        (2, _FEATURES + 1, _LANES))
    k = _make_kernel(info.num_cores, info.num_subcores)
    return k(table, idx, wb)
